# baseline (speedup 1.0000x reference)
import jax
import jax.numpy as jnp
from jax.experimental import pallas as pl
from jax.experimental.pallas import tpu as pltpu


def _qsample_body(t_ref, a_tbl, s_tbl, x0_ref, n_ref, xt_ref):
    i = pl.program_id(0)
    tt = t_ref[0, i]
    a = a_tbl[0, tt]
    s = s_tbl[0, tt]
    xt_ref[...] = a * x0_ref[...] + s * n_ref[...]


_SUB_BLK = 256


def kernel(x0, t, noise, sqrt_alphas_bar, sqrt_one_minus_alphas_bar):
    b = x0.shape[0]
    row = x0.size // b
    sub = row // 128
    x0f = x0.reshape(b, sub, 128)
    nf = noise.reshape(b, sub, 128)
    xt = pl.pallas_call(
        _qsample_body,
        grid=(b, sub // _SUB_BLK),
        in_specs=[
            pl.BlockSpec(memory_space=pltpu.SMEM),
            pl.BlockSpec(memory_space=pltpu.SMEM),
            pl.BlockSpec(memory_space=pltpu.SMEM),
            pl.BlockSpec((1, _SUB_BLK, 128), lambda i, j: (i, j, 0)),
            pl.BlockSpec((1, _SUB_BLK, 128), lambda i, j: (i, j, 0)),
        ],
        out_specs=pl.BlockSpec((1, _SUB_BLK, 128), lambda i, j: (i, j, 0)),
        out_shape=jax.ShapeDtypeStruct((b, sub, 128), jnp.float32),
    )(
        t.reshape(1, b).astype(jnp.int32),
        sqrt_alphas_bar.reshape(1, -1),
        sqrt_one_minus_alphas_bar.reshape(1, -1),
        x0f,
        nf,
    )
    return xt.reshape(x0.shape), noise


# speedup vs baseline: 1.0006x; 1.0006x over previous
import jax
import jax.numpy as jnp
from jax.experimental import pallas as pl
from jax.experimental.pallas import tpu as pltpu


def _qsample_body(t_ref, a_tbl, s_tbl, x0_ref, n_ref, xt_ref):
    i = pl.program_id(0)
    tt = t_ref[0, i]
    a = a_tbl[0, tt]
    s = s_tbl[0, tt]
    xt_ref[...] = a * x0_ref[...] + s * n_ref[...]


_SUB_BLK = 256


def kernel(x0, t, noise, sqrt_alphas_bar, sqrt_one_minus_alphas_bar):
    b = x0.shape[0]
    row = x0.size // b
    sub = row // 128
    x0f = x0.reshape(b, sub, 128)
    nf = noise.reshape(b, sub, 128)
    xt = pl.pallas_call(
        _qsample_body,
        grid=(b, sub // _SUB_BLK),
        in_specs=[
            pl.BlockSpec(memory_space=pltpu.SMEM),
            pl.BlockSpec(memory_space=pltpu.SMEM),
            pl.BlockSpec(memory_space=pltpu.SMEM),
            pl.BlockSpec((1, _SUB_BLK, 128), lambda i, j: (i, j, 0)),
            pl.BlockSpec((1, _SUB_BLK, 128), lambda i, j: (i, j, 0)),
        ],
        out_specs=pl.BlockSpec((1, _SUB_BLK, 128), lambda i, j: (i, j, 0)),
        out_shape=jax.ShapeDtypeStruct((b, sub, 128), jnp.float32),
        compiler_params=pltpu.CompilerParams(
            dimension_semantics=("parallel", "parallel"),
        ),
    )(
        t.reshape(1, b).astype(jnp.int32),
        sqrt_alphas_bar.reshape(1, -1),
        sqrt_one_minus_alphas_bar.reshape(1, -1),
        x0f,
        nf,
    )
    return xt.reshape(x0.shape), noise


# speedup vs baseline: 4.1259x; 4.1233x over previous
import jax
import jax.numpy as jnp
from jax.experimental import pallas as pl
from jax.experimental.pallas import tpu as pltpu


def _qsample_body(t_ref, a_tbl, s_tbl, x0_ref, n_ref, xt_ref):
    i = pl.program_id(0)
    tt = t_ref[0, i]
    a = a_tbl[0, tt]
    s = s_tbl[0, tt]
    xt_ref[...] = a * x0_ref[...] + s * n_ref[...]


_SUB_BLK = 256


def kernel(x0, t, noise, sqrt_alphas_bar, sqrt_one_minus_alphas_bar):
    b, c, h, w = x0.shape
    xt = pl.pallas_call(
        _qsample_body,
        grid=(b,),
        in_specs=[
            pl.BlockSpec(memory_space=pltpu.SMEM),
            pl.BlockSpec(memory_space=pltpu.SMEM),
            pl.BlockSpec(memory_space=pltpu.SMEM),
            pl.BlockSpec((1, c, h, w), lambda i: (i, 0, 0, 0)),
            pl.BlockSpec((1, c, h, w), lambda i: (i, 0, 0, 0)),
        ],
        out_specs=pl.BlockSpec((1, c, h, w), lambda i: (i, 0, 0, 0)),
        out_shape=jax.ShapeDtypeStruct((b, c, h, w), jnp.float32),
        compiler_params=pltpu.CompilerParams(
            dimension_semantics=("parallel",),
        ),
    )(
        t.reshape(1, b).astype(jnp.int32),
        sqrt_alphas_bar.reshape(1, -1),
        sqrt_one_minus_alphas_bar.reshape(1, -1),
        x0,
        noise,
    )
    return xt, noise


# speedup vs baseline: 5.1659x; 1.2521x over previous
import jax
import jax.numpy as jnp
from jax.experimental import pallas as pl
from jax.experimental.pallas import tpu as pltpu

_BB = 8


def _qsample_body(t_ref, a_tbl, s_tbl, x0_ref, n_ref, xt_ref):
    i = pl.program_id(0)
    for k in range(_BB):
        tt = t_ref[0, i * _BB + k]
        a = a_tbl[0, tt]
        s = s_tbl[0, tt]
        xt_ref[k] = a * x0_ref[k] + s * n_ref[k]


def kernel(x0, t, noise, sqrt_alphas_bar, sqrt_one_minus_alphas_bar):
    b, c, h, w = x0.shape
    xt = pl.pallas_call(
        _qsample_body,
        grid=(b // _BB,),
        in_specs=[
            pl.BlockSpec(memory_space=pltpu.SMEM),
            pl.BlockSpec(memory_space=pltpu.SMEM),
            pl.BlockSpec(memory_space=pltpu.SMEM),
            pl.BlockSpec((_BB, c, h, w), lambda i: (i, 0, 0, 0)),
            pl.BlockSpec((_BB, c, h, w), lambda i: (i, 0, 0, 0)),
        ],
        out_specs=pl.BlockSpec((_BB, c, h, w), lambda i: (i, 0, 0, 0)),
        out_shape=jax.ShapeDtypeStruct((b, c, h, w), jnp.float32),
        compiler_params=pltpu.CompilerParams(
            dimension_semantics=("parallel",),
        ),
    )(
        t.reshape(1, b).astype(jnp.int32),
        sqrt_alphas_bar.reshape(1, -1),
        sqrt_one_minus_alphas_bar.reshape(1, -1),
        x0,
        noise,
    )
    return xt, noise


# speedup vs baseline: 5.1729x; 1.0013x over previous
import jax
import jax.numpy as jnp
from jax.experimental import pallas as pl
from jax.experimental.pallas import tpu as pltpu

_BB = 4


def _qsample_body(t_ref, a_tbl, s_tbl, x0_ref, n_ref, xt_ref):
    i = pl.program_id(0)
    for k in range(_BB):
        tt = t_ref[0, i * _BB + k]
        a = a_tbl[0, tt]
        s = s_tbl[0, tt]
        xt_ref[k] = a * x0_ref[k] + s * n_ref[k]


def kernel(x0, t, noise, sqrt_alphas_bar, sqrt_one_minus_alphas_bar):
    b, c, h, w = x0.shape
    xt = pl.pallas_call(
        _qsample_body,
        grid=(b // _BB,),
        in_specs=[
            pl.BlockSpec(memory_space=pltpu.SMEM),
            pl.BlockSpec(memory_space=pltpu.SMEM),
            pl.BlockSpec(memory_space=pltpu.SMEM),
            pl.BlockSpec((_BB, c, h, w), lambda i: (i, 0, 0, 0)),
            pl.BlockSpec((_BB, c, h, w), lambda i: (i, 0, 0, 0)),
        ],
        out_specs=pl.BlockSpec((_BB, c, h, w), lambda i: (i, 0, 0, 0)),
        out_shape=jax.ShapeDtypeStruct((b, c, h, w), jnp.float32),
        compiler_params=pltpu.CompilerParams(
            dimension_semantics=("parallel",),
        ),
    )(
        t.reshape(1, b).astype(jnp.int32),
        sqrt_alphas_bar.reshape(1, -1),
        sqrt_one_minus_alphas_bar.reshape(1, -1),
        x0,
        noise,
    )
    return xt, noise
